# baseline (speedup 1.0000x reference)
import functools

import jax
import jax.numpy as jnp
from jax import lax
from jax.experimental import pallas as pl
from jax.experimental.pallas import tpu as pltpu
from jax.experimental.pallas import tpu_sc as plsc

NUM_EMB = 1000000
H = 64
BATCH = 4096
HIST = 200

NC = 2
NS = 16
NW = NC * NS

N = BATCH * HIST
M = N // NW
C = 128
SUB = C // 16
K = M // C
NBUF = 4
NPACK = 2
LAG = 3
T = K // NBUF


def _make_gather():
    mesh = plsc.VectorSubcoreMesh(core_axis_name="c", subcore_axis_name="s")

    @functools.partial(
        pl.kernel,
        mesh=mesh,
        out_type=jax.ShapeDtypeStruct((N // 2, 2 * H), jnp.float32),
        scratch_types=[
            pltpu.VMEM((K, C), jnp.int32),
            pltpu.VMEM((NBUF, C, 2 * H), jnp.float32),
            pltpu.VMEM((NPACK, C // 2, 2 * H), jnp.float32),
            pltpu.SemaphoreType.DMA((NBUF,)),
            pltpu.SemaphoreType.DMA((NPACK,)),
        ],
        compiler_params=pltpu.CompilerParams(needs_layout_passes=False),
    )
    def k(table_hbm, idx_hbm, out_hbm, idx_v, rows_v, pack_v, gsem, osem):
        wid = lax.axis_index("s") * NC + lax.axis_index("c")
        base2 = wid * (M // 2)
        pltpu.sync_copy(idx_hbm.at[pl.ds(wid * K, K)], idx_v)

        def gather_descs(j, slot):
            descs = []
            for u in range(SUB):
                vec = idx_v[j, pl.ds(u * 16, 16)] >> 1
                descs.append(pltpu.make_async_copy(
                    table_hbm.at[vec],
                    rows_v.at[slot, pl.ds(u * 16, 16)],
                    gsem.at[slot],
                ))
            return descs

        def store_desc(j, p):
            return pltpu.make_async_copy(
                pack_v.at[p],
                out_hbm.at[pl.ds(base2 + j * (C // 2), C // 2)],
                osem.at[p],
            )

        def compact(j, slot, p):
            rows2d = rows_v.at[slot]
            pack2d = pack_v.at[p]
            lanes = lax.iota(jnp.int32, 16)

            def body(g, carry):
                ids16 = idx_v[j, pl.ds(g * 16, 16)]
                scol0 = (ids16 & 1) * H
                srow = g * 16 + lanes
                drow = g * 8 + (lanes >> 1)
                dcol0 = (lanes & 1) * H
                for kk in range(H):
                    v = plsc.load_gather(rows2d, [srow, scol0 + kk])
                    plsc.store_scatter(pack2d, [drow, dcol0 + kk], v)
                return carry

            lax.fori_loop(0, C // 16, body, 0)

        def drain_compact_store(j2, b2, p):
            @pl.when(j2 >= NPACK)
            def _():
                store_desc(j2 - NPACK, p).wait()

            for d in gather_descs(j2, b2):
                d.wait()
            compact(j2, b2, p)
            store_desc(j2, p).start()

        def round_body(t, carry):
            for b in range(NBUF):
                j = t * NBUF + b
                for d in gather_descs(j, b):
                    d.start()

                j2 = j - LAG
                b2 = (b + NBUF - LAG) % NBUF
                p = (b + NBUF - LAG) % NPACK

                @pl.when(j2 >= 0)
                def _():
                    drain_compact_store(j2, b2, p)

            return carry

        lax.fori_loop(0, T, round_body, 0)

        for e in range(LAG):
            j2 = K - LAG + e
            drain_compact_store(j2, j2 % NBUF, j2 % NPACK)
        for p in range(NPACK):
            j = K - NPACK + p
            store_desc(j, j % NPACK).wait()

    return k


_gather = _make_gather()


@jax.jit
def kernel(input_ids, table):
    ids2 = input_ids.reshape(N // C, C).astype(jnp.int32)
    table2 = table.reshape(NUM_EMB // 2, 2 * H)
    out = _gather(table2, ids2)
    return out.reshape(BATCH, HIST, H)


# speedup vs baseline: 1.8494x; 1.8494x over previous
import functools

import jax
import jax.numpy as jnp
from jax import lax
from jax.experimental import pallas as pl
from jax.experimental.pallas import tpu as pltpu
from jax.experimental.pallas import tpu_sc as plsc

NUM_EMB = 1000000
H = 64
BATCH = 4096
HIST = 200

NC = 2
NS = 16
NW = NC * NS

N = BATCH * HIST
M = N // NW
C = 128
SUB = C // 16
K = M // C
NBUF = 4
NPACK = 2
LAG = 3
T = K // NBUF


def _make_gather():
    mesh = plsc.VectorSubcoreMesh(core_axis_name="c", subcore_axis_name="s")

    @functools.partial(
        pl.kernel,
        mesh=mesh,
        out_type=jax.ShapeDtypeStruct((N // 2, 2 * H), jnp.float32),
        scratch_types=[
            pltpu.VMEM((K, C), jnp.int32),
            pltpu.VMEM((NBUF, C, 2 * H), jnp.float32),
            pltpu.VMEM((NPACK, C // 2, 2 * H), jnp.float32),
            pltpu.SemaphoreType.DMA((NBUF,)),
            pltpu.SemaphoreType.DMA((NPACK,)),
        ],
        compiler_params=pltpu.CompilerParams(needs_layout_passes=False),
    )
    def k(table_hbm, idx_hbm, out_hbm, idx_v, rows_v, pack_v, gsem, osem):
        wid = lax.axis_index("s") * NC + lax.axis_index("c")
        base2 = wid * (M // 2)
        pltpu.sync_copy(idx_hbm.at[pl.ds(wid * K, K)], idx_v)

        def gather_descs(j, slot):
            descs = []
            for u in range(SUB):
                vec = idx_v[j, pl.ds(u * 16, 16)] >> 1
                descs.append(pltpu.make_async_copy(
                    table_hbm.at[vec],
                    rows_v.at[slot, pl.ds(u * 16, 16)],
                    gsem.at[slot],
                ))
            return descs

        def store_desc(j, p):
            return pltpu.make_async_copy(
                pack_v.at[p],
                out_hbm.at[pl.ds(base2 + j * (C // 2), C // 2)],
                osem.at[p],
            )

        def compact(j, slot, p):
            def body(i, carry):
                g16 = idx_v[j, pl.ds((i >> 3) * 16, 16)]
                l0 = (i & 7) * 2
                m0 = (g16.at[jnp.full((16,), 0, jnp.int32) + l0]
                      .get(mode="promise_in_bounds") & 1) == 1
                m1 = (g16.at[jnp.full((16,), 1, jnp.int32) + l0]
                      .get(mode="promise_in_bounds") & 1) == 1
                for q in range(H // 16):
                    a = rows_v[slot, 2 * i, pl.ds(q * 16, 16)]
                    bb = rows_v[slot, 2 * i, pl.ds(H + q * 16, 16)]
                    pack_v[p, i, pl.ds(q * 16, 16)] = jnp.where(m0, bb, a)
                for q in range(H // 16):
                    a = rows_v[slot, 2 * i + 1, pl.ds(q * 16, 16)]
                    bb = rows_v[slot, 2 * i + 1, pl.ds(H + q * 16, 16)]
                    pack_v[p, i, pl.ds(H + q * 16, 16)] = jnp.where(m1, bb, a)
                return carry

            lax.fori_loop(0, C // 2, body, 0)

        def drain_compact_store(j2, b2, p):
            @pl.when(j2 >= NPACK)
            def _():
                store_desc(j2 - NPACK, p).wait()

            for d in gather_descs(j2, b2):
                d.wait()
            compact(j2, b2, p)
            store_desc(j2, p).start()

        def round_body(t, carry):
            for b in range(NBUF):
                j = t * NBUF + b
                for d in gather_descs(j, b):
                    d.start()

                j2 = j - LAG
                b2 = (b + NBUF - LAG) % NBUF
                p = (b + NBUF - LAG) % NPACK

                @pl.when(j2 >= 0)
                def _():
                    drain_compact_store(j2, b2, p)

            return carry

        lax.fori_loop(0, T, round_body, 0)

        for e in range(LAG):
            j2 = K - LAG + e
            drain_compact_store(j2, j2 % NBUF, j2 % NPACK)
        for p in range(NPACK):
            j = K - NPACK + p
            store_desc(j, j % NPACK).wait()

    return k


_gather = _make_gather()


@jax.jit
def kernel(input_ids, table):
    ids2 = input_ids.reshape(N // C, C).astype(jnp.int32)
    table2 = table.reshape(NUM_EMB // 2, 2 * H)
    out = _gather(table2, ids2)
    return out.reshape(BATCH, HIST, H)


# speedup vs baseline: 2.4092x; 1.3027x over previous
import functools

import jax
import jax.numpy as jnp
from jax import lax
from jax.experimental import pallas as pl
from jax.experimental.pallas import tpu as pltpu
from jax.experimental.pallas import tpu_sc as plsc

NUM_EMB = 1000000
H = 64
BATCH = 4096
HIST = 200

NC = 2
NS = 16
NW = NC * NS

N = BATCH * HIST
M = N // NW
C = 128
K = M // C
NBUF = 8
LAG = 4
T = K // NBUF


def _make_gather():
    mesh = plsc.VectorSubcoreMesh(core_axis_name="c", subcore_axis_name="s")

    @functools.partial(
        pl.kernel,
        mesh=mesh,
        out_type=jax.ShapeDtypeStruct((N, H), jnp.float32),
        scratch_types=[
            pltpu.VMEM((M,), jnp.int32),
            pltpu.VMEM((NBUF, C, H), jnp.float32),
            pltpu.SemaphoreType.DMA((NBUF,)),
            pltpu.SemaphoreType.DMA((NBUF,)),
        ],
        compiler_params=pltpu.CompilerParams(use_tc_tiling_on_sc=False),
    )
    def k(table_hbm, idx_hbm, out_hbm, idx_v, rows_v, gsem, osem):
        wid = lax.axis_index("s") * NC + lax.axis_index("c")
        base = wid * M
        pltpu.sync_copy(idx_hbm.at[pl.ds(base, M)], idx_v)

        def gather_desc(j, slot):
            return pltpu.make_async_copy(
                table_hbm.at[idx_v.at[pl.ds(j * C, C)]],
                rows_v.at[slot],
                gsem.at[slot],
            )

        def store_desc(j, slot):
            return pltpu.make_async_copy(
                rows_v.at[slot],
                out_hbm.at[pl.ds(base + j * C, C)],
                osem.at[slot],
            )

        def round_body(t, carry):
            for b in range(NBUF):
                j = t * NBUF + b
                @pl.when(j >= NBUF)
                def _():
                    store_desc(j - NBUF, b).wait()

                gather_desc(j, b).start()

                j2 = j - LAG
                b2 = (b + NBUF - LAG) % NBUF

                @pl.when(j2 >= 0)
                def _():
                    gather_desc(j2, b2).wait()
                    store_desc(j2, b2).start()

            return carry

        lax.fori_loop(0, T, round_body, 0)

        for b in range(NBUF - LAG, NBUF):
            j2 = K - NBUF + b
            gather_desc(j2, b).wait()
            store_desc(j2, b).start()
        for b in range(NBUF):
            store_desc(K - NBUF + b, b).wait()

    return k


_gather = _make_gather()


@jax.jit
def kernel(input_ids, table):
    ids_flat = input_ids.reshape(-1).astype(jnp.int32)
    out = _gather(table, ids_flat)
    return out.reshape(BATCH, HIST, H)
